# Initial kernel scaffold; baseline (speedup 1.0000x reference)
import jax
import jax.numpy as jnp
from jax.experimental import pallas as pl


def kernel(labels, weight):
    raise NotImplementedError("write your pallas kernel here")


# speedup vs baseline: 4.0920x; 4.0920x over previous
import jax
import jax.numpy as jnp
from jax.experimental import pallas as pl

_B = 4096
_L = 200
_D = 32
_BLK = 128


def _body(labels_ref, w_ref, out_ref):
    mask = labels_ref[...] != 0
    w = w_ref[...]
    out_ref[...] = jnp.where(mask, w, 0.0)


def kernel(labels, weight):
    wslice = jax.lax.slice(weight, (1, 0), (1 + _L, _D)).reshape(1, _L, _D)
    labels3 = labels.reshape(_B, _L, 1)
    return pl.pallas_call(
        _body,
        grid=(_B // _BLK,),
        in_specs=[
            pl.BlockSpec((_BLK, _L, 1), lambda i: (i, 0, 0)),
            pl.BlockSpec((1, _L, _D), lambda i: (0, 0, 0)),
        ],
        out_specs=pl.BlockSpec((_BLK, _L, _D), lambda i: (i, 0, 0)),
        out_shape=jax.ShapeDtypeStruct((_B, _L, _D), jnp.float32),
    )(labels3, wslice)


# speedup vs baseline: 21.1989x; 5.1805x over previous
import jax
import jax.numpy as jnp
from jax.experimental import pallas as pl

_B = 4096
_L = 200
_D = 32
_BLK = 256


def _body(labels_ref, ew_ref, out_ref):
    m = (labels_ref[...] != 0).astype(jnp.float32)
    out_ref[...] = jax.lax.dot(m, ew_ref[...],
                               preferred_element_type=jnp.float32)


def kernel(labels, weight):
    wflat = jax.lax.slice(weight, (1, 0), (1 + _L, _D)).reshape(_L * _D)
    col = jnp.arange(_L * _D, dtype=jnp.int32) // _D
    onehot = (col[None, :] == jnp.arange(_L, dtype=jnp.int32)[:, None])
    ew = onehot.astype(jnp.float32) * wflat[None, :]
    out2 = pl.pallas_call(
        _body,
        grid=(_B // _BLK,),
        in_specs=[
            pl.BlockSpec((_BLK, _L), lambda i: (i, 0)),
            pl.BlockSpec((_L, _L * _D), lambda i: (0, 0)),
        ],
        out_specs=pl.BlockSpec((_BLK, _L * _D), lambda i: (i, 0)),
        out_shape=jax.ShapeDtypeStruct((_B, _L * _D), jnp.float32),
    )(labels, ew)
    return out2.reshape(_B, _L, _D)
